# baseline (speedup 1.0000x reference)
import functools

import jax
import jax.numpy as jnp
from jax import lax
from jax.experimental import pallas as pl
from jax.experimental.pallas import tpu as pltpu
from jax.experimental.pallas import tpu_sc as plsc

_N = 2048
_PRO = 1024
_F1, _F2, _F3 = 256, 128, 64
_E_TOTAL = 32768 + _N
_GAMMA = 0.03125

_NC, _NS = 2, 16
_EPT = _E_TOTAL // _NS
_ECH = _EPT // 16
_RB = 512
_PHASES = _N // _NC // _RB
_RPT = _RB // _NS
_DUMMY = _RB * _N


_ZW = 16384


def _counts_body(src_hbm, dst_hbm, c_hbm, src_v, dst_v, idx_v, ones_v,
                 zeros_v, shared, sem):
    cid = lax.axis_index("c")
    sid = lax.axis_index("s")
    ebase = sid * _EPT
    pltpu.sync_copy(src_hbm.at[pl.ds(ebase, _EPT)], src_v)
    pltpu.sync_copy(dst_hbm.at[pl.ds(ebase, _EPT)], dst_v)

    def _zfill(t, _):
        for u in range(8):
            zeros_v[pl.ds(t * 128 + u * 16, 16)] = jnp.zeros(
                (16,), jnp.float32)
        return _
    lax.fori_loop(0, _ZW // 128, _zfill, None)
    for j in range(17):
        for l in range(8):
            ones_v[j, pl.ds(l * 16, 16)] = jnp.ones((16,), jnp.float32)

    for ph in range(_PHASES):
        row_base = cid * (_N // _NC) + ph * _RB
        with jax.named_scope("zero_idx"):
            zc = [pltpu.async_copy(
                      zeros_v,
                      shared.at[pl.ds(sid * _RPT * _N + r * _ZW, _ZW)], sem)
                  for r in range(_RPT * _N // _ZW)]
            dummy = _DUMMY + (sid * 16 + lax.iota(jnp.int32, 16)) * 8
            for c in range(_ECH):
                j, l = divmod(c, 8)
                s16 = src_v[pl.ds(c * 16, 16)]
                d16 = dst_v[pl.ds(c * 16, 16)]
                inr = (d16 >= row_base) & (d16 < row_base + _RB)
                flat = jnp.where(inr, (d16 - row_base) * _N + s16, dummy)
                idx_v[j, pl.ds(l * 16, 16)] = flat
            for h in zc:
                h.wait()
        with jax.named_scope("bar1"):
            plsc.subcore_barrier()
        with jax.named_scope("scatter"):
            sc = [pltpu.async_copy(ones_v.at[j], shared.at[idx_v.at[j]],
                                   sem, add=True)
                  for j in range(17)]
            for h in sc:
                h.wait()
        with jax.named_scope("bar2"):
            plsc.subcore_barrier()
        with jax.named_scope("copyout"):
            out_row = row_base + sid * _RPT
            oc = [pltpu.async_copy(
                      shared.at[pl.ds((sid * _RPT + r) * _N, _N)],
                      c_hbm.at[out_row + r], sem)
                  for r in range(_RPT)]
            for h in oc:
                h.wait()
        with jax.named_scope("bar3"):
            plsc.subcore_barrier()


def _build_counts(src, dst):
    mesh = plsc.VectorSubcoreMesh(core_axis_name="c", subcore_axis_name="s")
    f = pl.kernel(
        _counts_body,
        out_type=jax.ShapeDtypeStruct((_N, _N), jnp.float32),
        mesh=mesh,
        scratch_types=[
            pltpu.VMEM((_EPT,), jnp.int32),
            pltpu.VMEM((_EPT,), jnp.int32),
            pltpu.VMEM((17, 128), jnp.int32),
            pltpu.VMEM((17, 128), jnp.float32),
            pltpu.VMEM((_ZW,), jnp.float32),
            pltpu.VMEM_SHARED((_RB * _N + 4096,), jnp.float32),
            pltpu.SemaphoreType.DMA,
        ],
    )
    return f(src, dst)


def _leaky(x):
    return jnp.where(x > 0, x, 0.2 * x)


def _mm_body(x_ref, w_ref, o_ref):
    o_ref[...] = jnp.dot(x_ref[...], w_ref[...],
                         preferred_element_type=jnp.float32)


def _matmul(x, w):
    n, k = x.shape
    f = w.shape[1]
    bm = 512
    return pl.pallas_call(
        _mm_body,
        grid=(n // bm,),
        in_specs=[pl.BlockSpec((bm, k), lambda i: (i, 0)),
                  pl.BlockSpec((k, f), lambda i: (0, 0))],
        out_specs=pl.BlockSpec((bm, f), lambda i: (i, 0)),
        out_shape=jax.ShapeDtypeStruct((n, f), jnp.float32),
    )(x, w)


def _att_body(c_ref, h_ref, hb_ref, as_ref, ad_ref, b_ref, o_ref):
    h = h_ref[...]
    hb = hb_ref[...]
    a_s = as_ref[...]
    a_d = ad_ref[...]
    nt = (((1,), (1,)), ((), ()))
    es_row = lax.dot_general(a_s, h, nt,
                             preferred_element_type=jnp.float32)
    ed_col = lax.dot_general(hb, a_d, nt,
                             preferred_element_type=jnp.float32)
    e = _leaky(ed_col + es_row)
    m_col = _leaky(jnp.max(es_row) + ed_col)
    p = c_ref[...] * jnp.exp(e - m_col)
    denom = jnp.sum(p, axis=1, keepdims=True)
    acc = jnp.dot(p, h, preferred_element_type=jnp.float32)
    o_ref[...] = jnp.maximum(acc / (denom + 1e-16) + b_ref[...], 0.0)


def _attention(c, h, a_s, a_d, b):
    f = h.shape[1]
    bm = 1024
    return pl.pallas_call(
        _att_body,
        grid=(_N // bm,),
        in_specs=[pl.BlockSpec((bm, _N), lambda i: (i, 0)),
                  pl.BlockSpec((_N, f), lambda i: (0, 0)),
                  pl.BlockSpec((bm, f), lambda i: (i, 0)),
                  pl.BlockSpec((1, f), lambda i: (0, 0)),
                  pl.BlockSpec((1, f), lambda i: (0, 0)),
                  pl.BlockSpec((1, f), lambda i: (0, 0))],
        out_specs=pl.BlockSpec((bm, f), lambda i: (i, 0)),
        out_shape=jax.ShapeDtypeStruct((_N, f), jnp.float32),
    )(c, h, h, a_s.reshape(1, f), a_d.reshape(1, f), b.reshape(1, f))


def _gip_body(y_ref, o_ref):
    y = y_ref[...]
    ymin = jnp.min(y, axis=1, keepdims=True)
    ymax = jnp.max(y, axis=1, keepdims=True)
    yn = (y - ymin) / (ymax - ymin + 1e-12)
    nt = (((1,), (1,)), ((), ()))
    k = lax.dot_general(yn, yn, nt, preferred_element_type=jnp.float32)
    yn2 = yn * yn
    ones = jnp.ones((1, yn.shape[1]), jnp.float32)
    di_row = lax.dot_general(ones, yn2, nt,
                             preferred_element_type=jnp.float32)
    di_col = lax.dot_general(yn2, ones, nt,
                             preferred_element_type=jnp.float32)
    scale = 1.0 / (jnp.mean(di_row) + 1e-12)
    d = (di_col + di_row - 2.0 * k) * scale
    o_ref[0] = jnp.exp(-_GAMMA * d)


def _gip(h):
    f = h.shape[1]
    return pl.pallas_call(
        _gip_body,
        grid=(2,),
        in_specs=[pl.BlockSpec((_PRO, f), lambda i: (i, 0))],
        out_specs=pl.BlockSpec((1, _PRO, _PRO), lambda i: (i, 0, 0)),
        out_shape=jax.ShapeDtypeStruct((2, _PRO, _PRO), jnp.float32),
    )(h)


def _comb_body(p1_ref, p2_ref, p3_ref, sim_ref, att_ref, al_ref, m_ref):
    a2 = att_ref[...]
    i = pl.program_id(0)
    sel = lambda j: jnp.where(i == 0, a2[0, j], a2[1, j])
    kc = (sel(0) * p1_ref[0] + sel(1) * p2_ref[0]
          + sel(2) * p3_ref[0] + sel(3) * sim_ref[0])
    d1 = jnp.sum(kc, axis=0, keepdims=True)
    pos = d1 > 0
    d5 = jnp.where(pos, lax.rsqrt(jnp.where(pos, d1, 1.0)), 0.0)
    nt = (((1,), (1,)), ((), ()))
    ones = jnp.ones((1, _PRO), jnp.float32)
    abar = lax.dot_general(ones, al_ref[0], nt,
                           preferred_element_type=jnp.float32) / _PRO
    v = d5 * abar
    kv = lax.dot_general(v, kc, nt, preferred_element_type=jnp.float32)
    m_ref[0] = jnp.where(pos, abar, 0.0) - d5 * kv


def _combine(pk1, pk2, pk3, sim, att, alph):
    blk3 = pl.BlockSpec((1, _PRO, _PRO), lambda i: (i, 0, 0))
    return pl.pallas_call(
        _comb_body,
        grid=(2,),
        in_specs=[blk3, blk3, blk3, blk3,
                  pl.BlockSpec((2, 4), lambda i: (0, 0)), blk3],
        out_specs=pl.BlockSpec((1, 1, _PRO), lambda i: (i, 0, 0)),
        out_shape=jax.ShapeDtypeStruct((2, 1, _PRO), jnp.float32),
    )(pk1, pk2, pk3, sim, att, alph).reshape(2, _PRO)


def _median_parts(m_row, m_col):
    le = (m_row <= m_col).astype(jnp.float32)
    cnt = jnp.sum(le, axis=1, keepdims=True)
    big = jnp.float32(3.0e38)
    v_lo = jnp.min(jnp.where(cnt >= _PRO // 2, m_col, big))
    v_hi = jnp.min(jnp.where(cnt >= _PRO // 2 + 1, m_col, big))
    return 0.5 * (v_lo + v_hi)


def _sigmoid(x):
    return 1.0 / (1.0 + jnp.exp(-x))


def _head_body(m_ref, cm1_ref, cm2_ref, cb1_ref, cb2_ref, f11_ref, f11b_ref,
               f12_ref, f12b_ref, f2_ref, f2b_ref, f3_ref, f3b_ref,
               f4_ref, f4b_ref, wm_ref, o_ref):
    nt = (((1,), (1,)), ((), ()))
    r = lax.broadcasted_iota(jnp.int32, (_PRO, _PRO), 0)
    c = lax.broadcasted_iota(jnp.int32, (_PRO, _PRO), 1)
    eye = (r == c).astype(jnp.float32)

    m1_row = m_ref[0:1, :]
    m2_row = m_ref[1:2, :]
    m1_col = lax.dot_general(eye, m1_row, nt,
                             preferred_element_type=jnp.float32)
    m2_col = lax.dot_general(eye, m2_row, nt,
                             preferred_element_type=jnp.float32)
    med1 = _median_parts(m1_row, m1_col)
    med2 = _median_parts(m2_row, m2_col)
    bf1_col = (m1_col > med1).astype(jnp.float32)
    bf2_row = (m2_row > med2).astype(jnp.float32)

    ones = jnp.ones((1, _PRO), jnp.float32)

    def _two_rows(cm_ref, cb_ref, fA_ref, fAb_ref):
        s = lax.dot_general(ones, cm_ref[...], nt,
                            preferred_element_type=jnp.float32)
        lo = jnp.maximum(cb_ref[...], 0.0)
        hi = jnp.maximum(s + cb_ref[...], 0.0)
        rows = jnp.concatenate([lo, hi], axis=0)
        o = jnp.maximum(lax.dot_general(rows, fA_ref[...], nt,
                        preferred_element_type=jnp.float32)
                        + fAb_ref[...], 0.0)
        o = jnp.maximum(lax.dot_general(o, f2_ref[...], nt,
                        preferred_element_type=jnp.float32)
                        + f2b_ref[...], 0.0)
        o = jnp.maximum(lax.dot_general(o, f3_ref[...], nt,
                        preferred_element_type=jnp.float32)
                        + f3b_ref[...], 0.0)
        o = _sigmoid(lax.dot_general(o, f4_ref[...], nt,
                     preferred_element_type=jnp.float32)
                     + f4b_ref[...])
        return o

    u1 = _two_rows(cm1_ref, cb1_ref, f11_ref, f11b_ref)
    u2 = _two_rows(cm2_ref, cb2_ref, f12_ref, f12b_ref)
    g = lax.dot_general(jnp.dot(u1, wm_ref[...],
                                preferred_element_type=jnp.float32),
                        u2, nt, preferred_element_type=jnp.float32)
    g00 = g[0, 0]
    g10 = g[1, 0]
    g01 = g[0, 1]
    g11 = g[1, 1]
    o_ref[...] = (g00 + bf1_col * (g10 - g00) + bf2_row * (g01 - g00)
                  + (bf1_col * bf2_row) * (g11 - g01 - g10 + g00))


def _head(m, cm1_W, cm1_b, cm2_W, cm2_b, fc11_W, fc11_b, fc12_W, fc12_b,
          fc2_W, fc2_b, fc3_W, fc3_b, fc4_W, fc4_b, weight_matrix):
    full = lambda s: pl.BlockSpec(s, lambda: tuple(0 for _ in s))
    args = [m, cm1_W, cm2_W, cm1_b.reshape(1, -1), cm2_b.reshape(1, -1),
            fc11_W, fc11_b.reshape(1, -1), fc12_W, fc12_b.reshape(1, -1),
            fc2_W, fc2_b.reshape(1, -1), fc3_W, fc3_b.reshape(1, -1),
            fc4_W, fc4_b.reshape(1, -1), weight_matrix]
    return pl.pallas_call(
        _head_body,
        in_specs=[full(a.shape) for a in args],
        out_specs=full((_PRO, _PRO)),
        out_shape=jax.ShapeDtypeStruct((_PRO, _PRO), jnp.float32),
    )(*args)


def _gat_layer(c, x, W, a_s, a_d, b):
    h = _matmul(x, W)
    return _attention(c, h, a_s, a_d, b)


def kernel(feature, W1, a1s, a1d, b1, W2, a2s, a2d, b2, W3, a3s, a3d, b3,
           att_m, att_d, pro_sim, drug_sim, alpha1, alpha2, cm1_W, cm1_b,
           cm2_W, cm2_b, fc11_W, fc11_b, fc12_W, fc12_b, fc2_W, fc2_b,
           fc3_W, fc3_b, fc4_W, fc4_b, weight_matrix, edge_index):
    loops = jnp.arange(_N, dtype=edge_index.dtype)
    src = jnp.concatenate([edge_index[0], loops])
    dst = jnp.concatenate([edge_index[1], loops])
    c = _build_counts(src, dst)

    H1 = _gat_layer(c, feature, W1, a1s, a1d, b1)
    pk1 = _gip(H1)
    H2 = _gat_layer(c, H1, W2, a2s, a2d, b2)
    pk2 = _gip(H2)
    H3 = _gat_layer(c, H2, W3, a3s, a3d, b3)
    pk3 = _gip(H3)

    sim = jnp.stack([pro_sim, drug_sim])
    att = jnp.concatenate([att_m, att_d], axis=0)
    alph = jnp.stack([alpha1, alpha2])
    m = _combine(pk1, pk2, pk3, sim, att, alph)

    return _head(m, cm1_W, cm1_b, cm2_W, cm2_b, fc11_W, fc11_b,
                 fc12_W, fc12_b, fc2_W, fc2_b, fc3_W, fc3_b,
                 fc4_W, fc4_b, weight_matrix)


# speedup vs baseline: 1.0248x; 1.0248x over previous
import functools

import jax
import jax.numpy as jnp
from jax import lax
from jax.experimental import pallas as pl
from jax.experimental.pallas import tpu as pltpu
from jax.experimental.pallas import tpu_sc as plsc

_N = 2048
_PRO = 1024
_F1, _F2, _F3 = 256, 128, 64
_E_TOTAL = 32768 + _N
_GAMMA = 0.03125

_NC, _NS = 2, 16
_EPT = _E_TOTAL // _NS
_ECH = _EPT // 16
_RB = 512
_PHASES = _N // _NC // _RB
_RPT = _RB // _NS
_DUMMY = _RB * _N


_ZW = 16384


def _counts_body(src_hbm, dst_hbm, c_hbm, src_v, dst_v, idx_v, ones_v,
                 zeros_v, shared, sem):
    cid = lax.axis_index("c")
    sid = lax.axis_index("s")
    ebase = sid * _EPT
    pltpu.sync_copy(src_hbm.at[pl.ds(ebase, _EPT)], src_v)
    pltpu.sync_copy(dst_hbm.at[pl.ds(ebase, _EPT)], dst_v)

    def _zfill(t, _):
        for u in range(8):
            zeros_v[pl.ds(t * 128 + u * 16, 16)] = jnp.zeros(
                (16,), jnp.float32)
        return _
    lax.fori_loop(0, _ZW // 128, _zfill, None)
    for j in range(17):
        for l in range(8):
            ones_v[j, pl.ds(l * 16, 16)] = jnp.ones((16,), jnp.float32)

    for ph in range(_PHASES):
        row_base = cid * (_N // _NC) + ph * _RB
        with jax.named_scope("zero_idx"):
            zc = [pltpu.async_copy(
                      zeros_v,
                      shared.at[pl.ds(sid * _RPT * _N + r * _ZW, _ZW)], sem)
                  for r in range(_RPT * _N // _ZW)]
            dummy = _DUMMY + (sid * 16 + lax.iota(jnp.int32, 16)) * 8
            for c in range(_ECH):
                j, l = divmod(c, 8)
                s16 = src_v[pl.ds(c * 16, 16)]
                d16 = dst_v[pl.ds(c * 16, 16)]
                inr = (d16 >= row_base) & (d16 < row_base + _RB)
                flat = jnp.where(inr, (d16 - row_base) * _N + s16, dummy)
                idx_v[j, pl.ds(l * 16, 16)] = flat
            for h in zc:
                h.wait()
        with jax.named_scope("bar1"):
            plsc.subcore_barrier()
        with jax.named_scope("scatter"):
            sc = [pltpu.async_copy(ones_v.at[j], shared.at[idx_v.at[j]],
                                   sem, add=True)
                  for j in range(17)]
            for h in sc:
                h.wait()
        with jax.named_scope("bar2"):
            plsc.subcore_barrier()
        with jax.named_scope("copyout"):
            out_row = row_base + sid * _RPT
            oc = [pltpu.async_copy(
                      shared.at[pl.ds((sid * _RPT + r) * _N, _N)],
                      c_hbm.at[out_row + r], sem)
                  for r in range(_RPT)]
            for h in oc:
                h.wait()
        with jax.named_scope("bar3"):
            plsc.subcore_barrier()


def _build_counts(src, dst):
    mesh = plsc.VectorSubcoreMesh(core_axis_name="c", subcore_axis_name="s")
    f = pl.kernel(
        _counts_body,
        out_type=jax.ShapeDtypeStruct((_N, _N), jnp.float32),
        mesh=mesh,
        scratch_types=[
            pltpu.VMEM((_EPT,), jnp.int32),
            pltpu.VMEM((_EPT,), jnp.int32),
            pltpu.VMEM((17, 128), jnp.int32),
            pltpu.VMEM((17, 128), jnp.float32),
            pltpu.VMEM((_ZW,), jnp.float32),
            pltpu.VMEM_SHARED((_RB * _N + 4096,), jnp.float32),
            pltpu.SemaphoreType.DMA,
        ],
    )
    return f(src, dst)


def _leaky(x):
    return jnp.where(x > 0, x, 0.2 * x)


def _mm_body(x_ref, w_ref, o_ref):
    o_ref[...] = jnp.dot(x_ref[...], w_ref[...],
                         preferred_element_type=jnp.float32)


def _matmul(x, w):
    n, k = x.shape
    f = w.shape[1]
    bm = 512
    return pl.pallas_call(
        _mm_body,
        grid=(n // bm,),
        in_specs=[pl.BlockSpec((bm, k), lambda i: (i, 0)),
                  pl.BlockSpec((k, f), lambda i: (0, 0))],
        out_specs=pl.BlockSpec((bm, f), lambda i: (i, 0)),
        out_shape=jax.ShapeDtypeStruct((n, f), jnp.float32),
    )(x, w)


def _att_body(c_ref, h_ref, hb_ref, as_ref, ad_ref, b_ref, o_ref):
    h = h_ref[...]
    hb = hb_ref[...]
    a_s = as_ref[...]
    a_d = ad_ref[...]
    nt = (((1,), (1,)), ((), ()))
    es_row = lax.dot_general(a_s, h, nt,
                             preferred_element_type=jnp.float32)
    ed_col = lax.dot_general(hb, a_d, nt,
                             preferred_element_type=jnp.float32)
    e = _leaky(ed_col + es_row)
    m_col = _leaky(jnp.max(es_row) + ed_col)
    p = c_ref[...] * jnp.exp(e - m_col)
    denom = jnp.sum(p, axis=1, keepdims=True)
    acc = jnp.dot(p, h, preferred_element_type=jnp.float32)
    o_ref[...] = jnp.maximum(acc / (denom + 1e-16) + b_ref[...], 0.0)


def _attention(c, h, a_s, a_d, b):
    f = h.shape[1]
    bm = 512
    return pl.pallas_call(
        _att_body,
        grid=(_N // bm,),
        in_specs=[pl.BlockSpec((bm, _N), lambda i: (i, 0)),
                  pl.BlockSpec((_N, f), lambda i: (0, 0)),
                  pl.BlockSpec((bm, f), lambda i: (i, 0)),
                  pl.BlockSpec((1, f), lambda i: (0, 0)),
                  pl.BlockSpec((1, f), lambda i: (0, 0)),
                  pl.BlockSpec((1, f), lambda i: (0, 0))],
        out_specs=pl.BlockSpec((bm, f), lambda i: (i, 0)),
        out_shape=jax.ShapeDtypeStruct((_N, f), jnp.float32),
    )(c, h, h, a_s.reshape(1, f), a_d.reshape(1, f), b.reshape(1, f))


def _gip_body(y_ref, o_ref):
    y = y_ref[...]
    ymin = jnp.min(y, axis=1, keepdims=True)
    ymax = jnp.max(y, axis=1, keepdims=True)
    yn = (y - ymin) / (ymax - ymin + 1e-12)
    nt = (((1,), (1,)), ((), ()))
    k = lax.dot_general(yn, yn, nt, preferred_element_type=jnp.float32)
    yn2 = yn * yn
    ones = jnp.ones((1, yn.shape[1]), jnp.float32)
    di_row = lax.dot_general(ones, yn2, nt,
                             preferred_element_type=jnp.float32)
    di_col = lax.dot_general(yn2, ones, nt,
                             preferred_element_type=jnp.float32)
    scale = 1.0 / (jnp.mean(di_row) + 1e-12)
    d = (di_col + di_row - 2.0 * k) * scale
    o_ref[0] = jnp.exp(-_GAMMA * d)


def _gip(h):
    f = h.shape[1]
    return pl.pallas_call(
        _gip_body,
        grid=(2,),
        in_specs=[pl.BlockSpec((_PRO, f), lambda i: (i, 0))],
        out_specs=pl.BlockSpec((1, _PRO, _PRO), lambda i: (i, 0, 0)),
        out_shape=jax.ShapeDtypeStruct((2, _PRO, _PRO), jnp.float32),
    )(h)


def _comb_body(p1_ref, p2_ref, p3_ref, sim_ref, att_ref, al_ref, m_ref):
    a2 = att_ref[...]
    i = pl.program_id(0)
    sel = lambda j: jnp.where(i == 0, a2[0, j], a2[1, j])
    kc = (sel(0) * p1_ref[0] + sel(1) * p2_ref[0]
          + sel(2) * p3_ref[0] + sel(3) * sim_ref[0])
    d1 = jnp.sum(kc, axis=0, keepdims=True)
    pos = d1 > 0
    d5 = jnp.where(pos, lax.rsqrt(jnp.where(pos, d1, 1.0)), 0.0)
    nt = (((1,), (1,)), ((), ()))
    ones = jnp.ones((1, _PRO), jnp.float32)
    abar = lax.dot_general(ones, al_ref[0], nt,
                           preferred_element_type=jnp.float32) / _PRO
    v = d5 * abar
    kv = lax.dot_general(v, kc, nt, preferred_element_type=jnp.float32)
    m_ref[0] = jnp.where(pos, abar, 0.0) - d5 * kv


def _combine(pk1, pk2, pk3, sim, att, alph):
    blk3 = pl.BlockSpec((1, _PRO, _PRO), lambda i: (i, 0, 0))
    return pl.pallas_call(
        _comb_body,
        grid=(2,),
        in_specs=[blk3, blk3, blk3, blk3,
                  pl.BlockSpec((2, 4), lambda i: (0, 0)), blk3],
        out_specs=pl.BlockSpec((1, 1, _PRO), lambda i: (i, 0, 0)),
        out_shape=jax.ShapeDtypeStruct((2, 1, _PRO), jnp.float32),
    )(pk1, pk2, pk3, sim, att, alph).reshape(2, _PRO)


def _median_parts(m_row, m_col):
    le = (m_row <= m_col).astype(jnp.float32)
    cnt = jnp.sum(le, axis=1, keepdims=True)
    big = jnp.float32(3.0e38)
    v_lo = jnp.min(jnp.where(cnt >= _PRO // 2, m_col, big))
    v_hi = jnp.min(jnp.where(cnt >= _PRO // 2 + 1, m_col, big))
    return 0.5 * (v_lo + v_hi)


def _sigmoid(x):
    return 1.0 / (1.0 + jnp.exp(-x))


def _head_body(m_ref, cm1_ref, cm2_ref, cb1_ref, cb2_ref, f11_ref, f11b_ref,
               f12_ref, f12b_ref, f2_ref, f2b_ref, f3_ref, f3b_ref,
               f4_ref, f4b_ref, wm_ref, o_ref):
    nt = (((1,), (1,)), ((), ()))
    r = lax.broadcasted_iota(jnp.int32, (_PRO, _PRO), 0)
    c = lax.broadcasted_iota(jnp.int32, (_PRO, _PRO), 1)
    eye = (r == c).astype(jnp.float32)

    m1_row = m_ref[0:1, :]
    m2_row = m_ref[1:2, :]
    m1_col = lax.dot_general(eye, m1_row, nt,
                             preferred_element_type=jnp.float32)
    m2_col = lax.dot_general(eye, m2_row, nt,
                             preferred_element_type=jnp.float32)
    med1 = _median_parts(m1_row, m1_col)
    med2 = _median_parts(m2_row, m2_col)
    bf1_col = (m1_col > med1).astype(jnp.float32)
    bf2_row = (m2_row > med2).astype(jnp.float32)

    ones = jnp.ones((1, _PRO), jnp.float32)

    def _two_rows(cm_ref, cb_ref, fA_ref, fAb_ref):
        s = lax.dot_general(ones, cm_ref[...], nt,
                            preferred_element_type=jnp.float32)
        lo = jnp.maximum(cb_ref[...], 0.0)
        hi = jnp.maximum(s + cb_ref[...], 0.0)
        rows = jnp.concatenate([lo, hi], axis=0)
        o = jnp.maximum(lax.dot_general(rows, fA_ref[...], nt,
                        preferred_element_type=jnp.float32)
                        + fAb_ref[...], 0.0)
        o = jnp.maximum(lax.dot_general(o, f2_ref[...], nt,
                        preferred_element_type=jnp.float32)
                        + f2b_ref[...], 0.0)
        o = jnp.maximum(lax.dot_general(o, f3_ref[...], nt,
                        preferred_element_type=jnp.float32)
                        + f3b_ref[...], 0.0)
        o = _sigmoid(lax.dot_general(o, f4_ref[...], nt,
                     preferred_element_type=jnp.float32)
                     + f4b_ref[...])
        return o

    u1 = _two_rows(cm1_ref, cb1_ref, f11_ref, f11b_ref)
    u2 = _two_rows(cm2_ref, cb2_ref, f12_ref, f12b_ref)
    g = lax.dot_general(jnp.dot(u1, wm_ref[...],
                                preferred_element_type=jnp.float32),
                        u2, nt, preferred_element_type=jnp.float32)
    g00 = g[0, 0]
    g10 = g[1, 0]
    g01 = g[0, 1]
    g11 = g[1, 1]
    o_ref[...] = (g00 + bf1_col * (g10 - g00) + bf2_row * (g01 - g00)
                  + (bf1_col * bf2_row) * (g11 - g01 - g10 + g00))


def _head(m, cm1_W, cm1_b, cm2_W, cm2_b, fc11_W, fc11_b, fc12_W, fc12_b,
          fc2_W, fc2_b, fc3_W, fc3_b, fc4_W, fc4_b, weight_matrix):
    full = lambda s: pl.BlockSpec(s, lambda: tuple(0 for _ in s))
    args = [m, cm1_W, cm2_W, cm1_b.reshape(1, -1), cm2_b.reshape(1, -1),
            fc11_W, fc11_b.reshape(1, -1), fc12_W, fc12_b.reshape(1, -1),
            fc2_W, fc2_b.reshape(1, -1), fc3_W, fc3_b.reshape(1, -1),
            fc4_W, fc4_b.reshape(1, -1), weight_matrix]
    return pl.pallas_call(
        _head_body,
        in_specs=[full(a.shape) for a in args],
        out_specs=full((_PRO, _PRO)),
        out_shape=jax.ShapeDtypeStruct((_PRO, _PRO), jnp.float32),
    )(*args)


def _gat_layer(c, x, W, a_s, a_d, b):
    h = _matmul(x, W)
    return _attention(c, h, a_s, a_d, b)


def kernel(feature, W1, a1s, a1d, b1, W2, a2s, a2d, b2, W3, a3s, a3d, b3,
           att_m, att_d, pro_sim, drug_sim, alpha1, alpha2, cm1_W, cm1_b,
           cm2_W, cm2_b, fc11_W, fc11_b, fc12_W, fc12_b, fc2_W, fc2_b,
           fc3_W, fc3_b, fc4_W, fc4_b, weight_matrix, edge_index):
    loops = jnp.arange(_N, dtype=edge_index.dtype)
    src = jnp.concatenate([edge_index[0], loops])
    dst = jnp.concatenate([edge_index[1], loops])
    c = _build_counts(src, dst)

    H1 = _gat_layer(c, feature, W1, a1s, a1d, b1)
    pk1 = _gip(H1)
    H2 = _gat_layer(c, H1, W2, a2s, a2d, b2)
    pk2 = _gip(H2)
    H3 = _gat_layer(c, H2, W3, a3s, a3d, b3)
    pk3 = _gip(H3)

    sim = jnp.stack([pro_sim, drug_sim])
    att = jnp.concatenate([att_m, att_d], axis=0)
    alph = jnp.stack([alpha1, alpha2])
    m = _combine(pk1, pk2, pk3, sim, att, alph)

    return _head(m, cm1_W, cm1_b, cm2_W, cm2_b, fc11_W, fc11_b,
                 fc12_W, fc12_b, fc2_W, fc2_b, fc3_W, fc3_b,
                 fc4_W, fc4_b, weight_matrix)
